# baseline (speedup 1.0000x reference)
import jax
import jax.numpy as jnp
from jax import lax
from jax.experimental import pallas as pl
from jax.experimental.pallas import tpu as pltpu
from jax.experimental.pallas import tpu_sc as plsc

N = 10000
E = 160000
D = 256
G = 64

NC = 2
NS = 16
NW = NC * NS
ROWS = 312
ROWS_LAST = N - (NW - 1) * ROWS
ACC_ROWS = 336
TRASH = 328
CAP = 7680
SCN = 8000
BLK = 64
PADB = 128

_PERM = []
for _p in range(D // 32):
    for _i in range(16):
        _PERM += [32 * _p + _i, 32 * _p + 16 + _i]

_mesh = plsc.VectorSubcoreMesh(core_axis_name="c", subcore_axis_name="s")


def _part_body(src_hbm, dst_hbm, w_hbm, srcl_hbm, dstl_hbm, wl_hbm, cnt_hbm,
               s_sc, d_sc, w_sc, ls_v, ld_v, lw_v, cb_v):
    c = lax.axis_index("c")
    s = lax.axis_index("s")
    wid = c * NS + s
    lo = wid * ROWS
    hi = jnp.where(wid == NW - 1, N, lo + ROWS)
    lo_v = jnp.full((16,), lo, jnp.int32)
    hi_v = jnp.full((16,), hi, jnp.int32)

    def _chunk(ch, off):
        base = ch * SCN
        pltpu.sync_copy(src_hbm.at[pl.ds(base, SCN)], s_sc)
        pltpu.sync_copy(dst_hbm.at[pl.ds(base, SCN)], d_sc)
        pltpu.sync_copy(w_hbm.at[pl.ds(base, SCN)], w_sc)

        def _vec(i, off):
            sl = pl.ds(i * 16, 16)
            d16 = d_sc[sl]
            mask = (d16 >= lo_v) & (d16 < hi_v)
            inc = plsc.cumsum(mask.astype(jnp.int32))
            pos = off + inc - 1
            plsc.store_scatter(ld_v, [pos], d16 - lo_v, mask=mask)
            plsc.store_scatter(ls_v, [pos], s_sc[sl], mask=mask)
            plsc.store_scatter(lw_v, [pos], w_sc[sl], mask=mask)
            return off + inc[15]

        return lax.fori_loop(0, SCN // 16, _vec, off)

    cnt = lax.fori_loop(0, E // SCN, _chunk, jnp.int32(0))

    zi = jnp.zeros((16,), jnp.int32)
    zf = jnp.zeros((16,), jnp.float32)
    tv = jnp.full((16,), TRASH, jnp.int32)
    for t in range(PADB // 16):
        sl = pl.ds(cnt + t * 16, 16)
        ls_v[sl] = zi
        ld_v[sl] = tv
        lw_v[sl] = zf
    cntp = ((cnt + PADB - 1) // PADB) * PADB

    pltpu.sync_copy(ls_v.at[pl.ds(0, CAP)], srcl_hbm.at[pl.ds(wid * CAP, CAP)])
    pltpu.sync_copy(ld_v.at[pl.ds(0, CAP)], dstl_hbm.at[pl.ds(wid * CAP, CAP)])
    pltpu.sync_copy(lw_v.at[pl.ds(0, CAP)], wl_hbm.at[pl.ds(wid * CAP, CAP)])

    cb_v[pl.ds(0, 16)] = jnp.full((16,), cntp, jnp.int32)
    pltpu.sync_copy(cb_v.at[pl.ds(0, 8)], cnt_hbm.at[pl.ds(wid * 8, 8)])


_part_call = pl.kernel(
    _part_body,
    out_type=(jax.ShapeDtypeStruct((NW * CAP,), jnp.int32),
              jax.ShapeDtypeStruct((NW * CAP,), jnp.int32),
              jax.ShapeDtypeStruct((NW * CAP,), jnp.float32),
              jax.ShapeDtypeStruct((NW * 8,), jnp.int32)),
    mesh=_mesh,
    compiler_params=pltpu.CompilerParams(needs_layout_passes=False),
    scratch_types=[
        pltpu.VMEM((SCN,), jnp.int32),
        pltpu.VMEM((SCN,), jnp.int32),
        pltpu.VMEM((SCN,), jnp.float32),
        pltpu.VMEM((CAP + PADB,), jnp.int32),
        pltpu.VMEM((CAP + PADB,), jnp.int32),
        pltpu.VMEM((CAP + PADB,), jnp.float32),
        pltpu.VMEM((16,), jnp.int32),
    ],
)


def _agg_body(gb_hbm, srcl_hbm, dstl_hbm, wl_hbm, cnt_hbm, s_hbm,
              ls_v, ld_v, lw_v, cv_v, rows_v, acc_v, sem0, sem1):
    c = lax.axis_index("c")
    s = lax.axis_index("s")
    wid = c * NS + s
    sems = (sem0, sem1)

    pltpu.sync_copy(srcl_hbm.at[pl.ds(wid * CAP, CAP)],
                    ls_v.at[pl.ds(0, CAP)])
    pltpu.sync_copy(dstl_hbm.at[pl.ds(wid * CAP, CAP)],
                    ld_v.at[pl.ds(0, CAP)])
    pltpu.sync_copy(wl_hbm.at[pl.ds(wid * CAP, CAP)],
                    lw_v.at[pl.ds(0, CAP)])
    pltpu.sync_copy(cnt_hbm, cv_v.at[pl.ds(0, NW * 8)])
    nb = cv_v[pl.ds(wid * 8, 16)][0] // BLK

    zf = jnp.zeros((16,), jnp.float32)

    def _zero(i, _):
        for jj in range(D // 16):
            acc_v[i, pl.ds(jj * 16, 16)] = zf
        return 0

    lax.fori_loop(0, ACC_ROWS, _zero, 0)

    himask = jnp.full((16,), -65536, jnp.int32)

    def _start(b, par):
        pltpu.async_copy(gb_hbm.at[ls_v.at[pl.ds(b * BLK, BLK)]],
                         rows_v.at[par], sems[par])

    def _wait(b, par):
        pltpu.make_async_copy(gb_hbm.at[ls_v.at[pl.ds(b * BLK, BLK)]],
                              rows_v.at[par], sems[par]).wait()

    def _proc(b, par):
        def _group(t, _):
            e0 = b * BLK + t * 16
            w16 = lw_v[pl.ds(e0, 16)]
            d16 = ld_v[pl.ds(e0, 16)]
            for l in range(16):
                wv = jnp.full((16,), w16[l], jnp.float32)
                dl = d16[l]
                k = t * 16 + l
                for cc in range(D // 32):
                    u = rows_v[par, k, pl.ds(cc * 16, 16)]
                    flo = plsc.bitcast(u << 16, jnp.float32)
                    fhi = plsc.bitcast(u & himask, jnp.float32)
                    col = cc * 32
                    sl0 = pl.ds(col, 16)
                    sl1 = pl.ds(col + 16, 16)
                    acc_v[dl, sl0] = acc_v[dl, sl0] + flo * wv
                    acc_v[dl, sl1] = acc_v[dl, sl1] + fhi * wv
            return 0

        lax.fori_loop(0, BLK // 16, _group, 0)

    @pl.when(nb > 0)
    def _():
        _start(0, 0)

    def _pair(pp, _):
        b0 = pp * 2
        _start(b0 + 1, 1)
        _wait(b0, 0)
        _proc(b0, 0)

        @pl.when(b0 + 2 < nb)
        def _():
            _start(b0 + 2, 0)

        _wait(b0 + 1, 1)
        _proc(b0 + 1, 1)
        return 0

    lax.fori_loop(0, nb // 2, _pair, 0)

    def _wb(off, n):
        pltpu.sync_copy(acc_v.at[pl.ds(off, n)],
                        s_hbm.at[pl.ds(wid * ROWS + off, n)])

    _wb(0, 128)
    _wb(128, 128)

    @pl.when(wid < NW - 1)
    def _():
        _wb(256, ROWS - 256)

    @pl.when(wid == NW - 1)
    def _():
        _wb(256, ROWS_LAST - 256)


_agg_full = pl.kernel(
    _agg_body,
    out_type=jax.ShapeDtypeStruct((N, D), jnp.float32),
    mesh=_mesh,
    compiler_params=pltpu.CompilerParams(needs_layout_passes=False),
    scratch_types=[
        pltpu.VMEM((CAP + 16,), jnp.int32),
        pltpu.VMEM((CAP + 16,), jnp.int32),
        pltpu.VMEM((CAP + 16,), jnp.float32),
        pltpu.VMEM((NW * 8 + 16,), jnp.int32),
        pltpu.VMEM((2, BLK, D // 2), jnp.int32),
        pltpu.VMEM((ACC_ROWS, D), jnp.float32),
        pltpu.SemaphoreType.DMA,
        pltpu.SemaphoreType.DMA,
    ],
)


def _to_bf16_table(g):
    gb = g.astype(jnp.bfloat16)[:, jnp.array(_PERM, dtype=jnp.int32)]
    return lax.bitcast_convert_type(gb.reshape(N, D // 2, 2), jnp.int32)


def _deg_body(dstl_hbm, wl_hbm, cnt_hbm, deg_hbm, ld_v, lw_v, cv_v, acc_v):
    c = lax.axis_index("c")
    s = lax.axis_index("s")
    wid = c * NS + s

    pltpu.sync_copy(dstl_hbm.at[pl.ds(wid * CAP, CAP)], ld_v.at[pl.ds(0, CAP)])
    pltpu.sync_copy(wl_hbm.at[pl.ds(wid * CAP, CAP)], lw_v.at[pl.ds(0, CAP)])
    pltpu.sync_copy(cnt_hbm, cv_v.at[pl.ds(0, NW * 8)])
    ng = cv_v[pl.ds(wid * 8, 16)][0] // 16

    zf = jnp.zeros((16,), jnp.float32)

    def _zero(i, _):
        acc_v[i, pl.ds(0, 16)] = zf
        return 0

    lax.fori_loop(0, ACC_ROWS, _zero, 0)

    def _group(t, _):
        w16 = lw_v[pl.ds(t * 16, 16)]
        d16 = ld_v[pl.ds(t * 16, 16)]
        for l in range(16):
            wv = jnp.full((16,), w16[l], jnp.float32)
            dl = d16[l]
            acc_v[dl, pl.ds(0, 16)] = acc_v[dl, pl.ds(0, 16)] + wv
        return 0

    lax.fori_loop(0, ng, _group, 0)

    def _wb(off, n):
        pltpu.sync_copy(acc_v.at[pl.ds(off, n)],
                        deg_hbm.at[pl.ds(wid * ROWS + off, n)])

    _wb(0, 128)
    _wb(128, 128)

    @pl.when(wid < NW - 1)
    def _():
        _wb(256, ROWS - 256)

    @pl.when(wid == NW - 1)
    def _():
        _wb(256, ROWS_LAST - 256)


_deg_call = pl.kernel(
    _deg_body,
    out_type=jax.ShapeDtypeStruct((N, 16), jnp.float32),
    mesh=_mesh,
    compiler_params=pltpu.CompilerParams(needs_layout_passes=False),
    scratch_types=[
        pltpu.VMEM((CAP + 16,), jnp.int32),
        pltpu.VMEM((CAP + 16,), jnp.float32),
        pltpu.VMEM((NW * 8 + 16,), jnp.int32),
        pltpu.VMEM((ACC_ROWS, 16), jnp.float32),
    ],
)


def _tc1_body(x_ref, sdeg_ref, w1_ref, dinv_ref, g_ref):
    deg = sdeg_ref[...][:, 0:1] + 1.0
    dinv = lax.rsqrt(deg)
    dinv_ref[...] = dinv
    h = jnp.dot(x_ref[...], w1_ref[...], preferred_element_type=jnp.float32)
    g_ref[...] = dinv * h


_tc1 = pl.pallas_call(
    _tc1_body,
    out_shape=(jax.ShapeDtypeStruct((N, 1), jnp.float32),
               jax.ShapeDtypeStruct((N, D), jnp.float32)),
)


def _bn_rows(h, gam, bet):
    mu = jnp.mean(h, axis=0, keepdims=True)
    var = jnp.mean(h * h, axis=0, keepdims=True) - mu * mu
    return (h - mu) * lax.rsqrt(var + 1e-5) * gam + bet


def _tc_mid_body(s_ref, g_ref, dinv_ref, b_ref, gam_ref, bet_ref, wn_ref,
                 gn_ref):
    h = dinv_ref[...] * (s_ref[...] + g_ref[...]) + b_ref[...]
    h = _bn_rows(h, gam_ref[...], bet_ref[...])
    h = jnp.maximum(h, 0.0)
    hw = jnp.dot(h, wn_ref[...], preferred_element_type=jnp.float32)
    gn_ref[...] = dinv_ref[...] * hw


_tc_mid = pl.pallas_call(
    _tc_mid_body,
    out_shape=jax.ShapeDtypeStruct((N, D), jnp.float32),
)


def _tc_final_body(s_ref, g_ref, dinv_ref, b_ref, gam_ref, bet_ref,
                   batch_ref, out_ref):
    h = dinv_ref[...] * (s_ref[...] + g_ref[...]) + b_ref[...]
    h = _bn_rows(h, gam_ref[...], bet_ref[...])
    gid = lax.broadcasted_iota(jnp.int32, (N, G), 1)
    oh = (batch_ref[...] == gid).astype(jnp.float32)
    sums = lax.dot_general(oh, h, (((0,), (0,)), ((), ())),
                           preferred_element_type=jnp.float32)
    cnt = jnp.sum(oh, axis=0, keepdims=True)
    inv = 1.0 / jnp.maximum(cnt, 1.0)
    sums = sums * lax.dot_general(inv, jnp.ones((1, D), jnp.float32),
                                  (((0,), (0,)), ((), ())),
                                  preferred_element_type=jnp.float32)
    out_ref[...] = sums


_tc_final = pl.pallas_call(
    _tc_final_body,
    out_shape=jax.ShapeDtypeStruct((G, D), jnp.float32),
)


def kernel(x, edge_index, edge_weight, batch, W1, b1, W2, b2, W3, b3,
           g1, be1, g2, be2, g3, be3):
    src = edge_index[0]
    dst = edge_index[1]
    b1r = b1.reshape(1, D); b2r = b2.reshape(1, D); b3r = b3.reshape(1, D)
    g1r = g1.reshape(1, D); g2r = g2.reshape(1, D); g3r = g3.reshape(1, D)
    be1r = be1.reshape(1, D); be2r = be2.reshape(1, D); be3r = be3.reshape(1, D)

    srcl, dstl, wl, cnts = _part_call(src, dst, edge_weight)
    sdeg = _deg_call(dstl, wl, cnts)
    dinv, g = _tc1(x, sdeg, W1)

    S = _agg_full(_to_bf16_table(g), srcl, dstl, wl, cnts)
    g = _tc_mid(S, g, dinv, b1r, g1r, be1r, W1)

    S = _agg_full(_to_bf16_table(g), srcl, dstl, wl, cnts)
    g = _tc_mid(S, g, dinv, b1r, g1r, be1r, W2)

    S = _agg_full(_to_bf16_table(g), srcl, dstl, wl, cnts)
    g = _tc_mid(S, g, dinv, b2r, g2r, be2r, W3)

    S = _agg_full(_to_bf16_table(g), srcl, dstl, wl, cnts)
    return _tc_final(S, g, dinv, b3r, g3r, be3r, batch.reshape(N, 1))


# speedup vs baseline: 1.2370x; 1.2370x over previous
import jax
import jax.numpy as jnp
from jax import lax
from jax.experimental import pallas as pl
from jax.experimental.pallas import tpu as pltpu
from jax.experimental.pallas import tpu_sc as plsc

N = 10000
E = 160000
D = 256
G = 64

NC = 2
NS = 16
NW = NC * NS
ROWS = 312
ROWS_LAST = N - (NW - 1) * ROWS
ACC_ROWS = 336
TRASH = 328
CAP = 7680
SCN = 8000
BLK = 128
PADB = 128

_PERM = []
for _p in range(D // 32):
    for _i in range(16):
        _PERM += [32 * _p + _i, 32 * _p + 16 + _i]

_mesh = plsc.VectorSubcoreMesh(core_axis_name="c", subcore_axis_name="s")


def _part_body(src_hbm, dst_hbm, w_hbm, srcl_hbm, dstl_hbm, wl_hbm, cnt_hbm,
               s_sc, d_sc, w_sc, ls_v, ld_v, lw_v, cb_v):
    c = lax.axis_index("c")
    s = lax.axis_index("s")
    wid = c * NS + s
    lo = wid * ROWS
    hi = jnp.where(wid == NW - 1, N, lo + ROWS)
    lo_v = jnp.full((16,), lo, jnp.int32)
    hi_v = jnp.full((16,), hi, jnp.int32)

    def _chunk(ch, off):
        base = ch * SCN
        pltpu.sync_copy(src_hbm.at[pl.ds(base, SCN)], s_sc)
        pltpu.sync_copy(dst_hbm.at[pl.ds(base, SCN)], d_sc)
        pltpu.sync_copy(w_hbm.at[pl.ds(base, SCN)], w_sc)

        def _vec(i, off):
            sl = pl.ds(i * 16, 16)
            d16 = d_sc[sl]
            mask = (d16 >= lo_v) & (d16 < hi_v)
            inc = plsc.cumsum(mask.astype(jnp.int32))
            pos = off + inc - 1
            plsc.store_scatter(ld_v, [pos], d16 - lo_v, mask=mask)
            plsc.store_scatter(ls_v, [pos], s_sc[sl], mask=mask)
            plsc.store_scatter(lw_v, [pos], w_sc[sl], mask=mask)
            return off + inc[15]

        return lax.fori_loop(0, SCN // 16, _vec, off)

    cnt = lax.fori_loop(0, E // SCN, _chunk, jnp.int32(0))

    zi = jnp.zeros((16,), jnp.int32)
    zf = jnp.zeros((16,), jnp.float32)
    tv = jnp.full((16,), TRASH, jnp.int32)
    for t in range(PADB // 16):
        sl = pl.ds(cnt + t * 16, 16)
        ls_v[sl] = zi
        ld_v[sl] = tv
        lw_v[sl] = zf
    cntp = ((cnt + PADB - 1) // PADB) * PADB

    pltpu.sync_copy(ls_v.at[pl.ds(0, CAP)], srcl_hbm.at[pl.ds(wid * CAP, CAP)])
    pltpu.sync_copy(ld_v.at[pl.ds(0, CAP)], dstl_hbm.at[pl.ds(wid * CAP, CAP)])
    pltpu.sync_copy(lw_v.at[pl.ds(0, CAP)], wl_hbm.at[pl.ds(wid * CAP, CAP)])

    cb_v[pl.ds(0, 16)] = jnp.full((16,), cntp, jnp.int32)
    pltpu.sync_copy(cb_v.at[pl.ds(0, 8)], cnt_hbm.at[pl.ds(wid * 8, 8)])


_part_call = pl.kernel(
    _part_body,
    out_type=(jax.ShapeDtypeStruct((NW * CAP,), jnp.int32),
              jax.ShapeDtypeStruct((NW * CAP,), jnp.int32),
              jax.ShapeDtypeStruct((NW * CAP,), jnp.float32),
              jax.ShapeDtypeStruct((NW * 8,), jnp.int32)),
    mesh=_mesh,
    compiler_params=pltpu.CompilerParams(needs_layout_passes=False),
    scratch_types=[
        pltpu.VMEM((SCN,), jnp.int32),
        pltpu.VMEM((SCN,), jnp.int32),
        pltpu.VMEM((SCN,), jnp.float32),
        pltpu.VMEM((CAP + PADB,), jnp.int32),
        pltpu.VMEM((CAP + PADB,), jnp.int32),
        pltpu.VMEM((CAP + PADB,), jnp.float32),
        pltpu.VMEM((16,), jnp.int32),
    ],
)


def _agg_body(gb_hbm, srcl_hbm, dstl_hbm, wl_hbm, cnt_hbm, s_hbm,
              ls_v, ld_v, lw_v, cv_v, rows_v, acc_v, sem):
    c = lax.axis_index("c")
    s = lax.axis_index("s")
    wid = c * NS + s

    pltpu.sync_copy(srcl_hbm.at[pl.ds(wid * CAP, CAP)],
                    ls_v.at[pl.ds(0, CAP)])
    pltpu.sync_copy(dstl_hbm.at[pl.ds(wid * CAP, CAP)],
                    ld_v.at[pl.ds(0, CAP)])
    pltpu.sync_copy(wl_hbm.at[pl.ds(wid * CAP, CAP)],
                    lw_v.at[pl.ds(0, CAP)])
    pltpu.sync_copy(cnt_hbm, cv_v.at[pl.ds(0, NW * 8)])
    nb = cv_v[pl.ds(wid * 8, 16)][0] // BLK

    zf = jnp.zeros((16,), jnp.float32)

    def _zero(i, _):
        for jj in range(D // 16):
            acc_v[i, pl.ds(jj * 16, 16)] = zf
        return 0

    lax.fori_loop(0, ACC_ROWS, _zero, 0)

    himask = jnp.full((16,), -65536, jnp.int32)

    def _block(b, _):
        pltpu.async_copy(gb_hbm.at[ls_v.at[pl.ds(b * BLK, BLK)]],
                         rows_v, sem).wait()

        def _group(t, _):
            e0 = b * BLK + t * 16
            w16 = lw_v[pl.ds(e0, 16)]
            d16 = ld_v[pl.ds(e0, 16)]
            for l in range(16):
                wv = jnp.full((16,), w16[l], jnp.float32)
                dl = d16[l]
                k = t * 16 + l
                for cc in range(D // 32):
                    u = rows_v[k, pl.ds(cc * 16, 16)]
                    flo = plsc.bitcast(u << 16, jnp.float32)
                    fhi = plsc.bitcast(u & himask, jnp.float32)
                    col = cc * 32
                    sl0 = pl.ds(col, 16)
                    sl1 = pl.ds(col + 16, 16)
                    acc_v[dl, sl0] = acc_v[dl, sl0] + flo * wv
                    acc_v[dl, sl1] = acc_v[dl, sl1] + fhi * wv
            return 0

        lax.fori_loop(0, BLK // 16, _group, 0)
        return 0

    lax.fori_loop(0, nb, _block, 0)

    def _wb(off, n):
        pltpu.sync_copy(acc_v.at[pl.ds(off, n)],
                        s_hbm.at[pl.ds(wid * ROWS + off, n)])

    _wb(0, 128)
    _wb(128, 128)

    @pl.when(wid < NW - 1)
    def _():
        _wb(256, ROWS - 256)

    @pl.when(wid == NW - 1)
    def _():
        _wb(256, ROWS_LAST - 256)


_agg_full = pl.kernel(
    _agg_body,
    out_type=jax.ShapeDtypeStruct((N, D), jnp.float32),
    mesh=_mesh,
    compiler_params=pltpu.CompilerParams(needs_layout_passes=False),
    scratch_types=[
        pltpu.VMEM((CAP + 16,), jnp.int32),
        pltpu.VMEM((CAP + 16,), jnp.int32),
        pltpu.VMEM((CAP + 16,), jnp.float32),
        pltpu.VMEM((NW * 8 + 16,), jnp.int32),
        pltpu.VMEM((BLK, D // 2), jnp.int32),
        pltpu.VMEM((ACC_ROWS, D), jnp.float32),
        pltpu.SemaphoreType.DMA,
    ],
)


def _to_bf16_table(g):
    gb = g.astype(jnp.bfloat16)[:, jnp.array(_PERM, dtype=jnp.int32)]
    return lax.bitcast_convert_type(gb.reshape(N, D // 2, 2), jnp.int32)


def _deg_body(dstl_hbm, wl_hbm, cnt_hbm, deg_hbm, ld_v, lw_v, cv_v, acc_v):
    c = lax.axis_index("c")
    s = lax.axis_index("s")
    wid = c * NS + s

    pltpu.sync_copy(dstl_hbm.at[pl.ds(wid * CAP, CAP)], ld_v.at[pl.ds(0, CAP)])
    pltpu.sync_copy(wl_hbm.at[pl.ds(wid * CAP, CAP)], lw_v.at[pl.ds(0, CAP)])
    pltpu.sync_copy(cnt_hbm, cv_v.at[pl.ds(0, NW * 8)])
    ng = cv_v[pl.ds(wid * 8, 16)][0] // 16

    zf = jnp.zeros((16,), jnp.float32)

    def _zero(i, _):
        acc_v[i, pl.ds(0, 16)] = zf
        return 0

    lax.fori_loop(0, ACC_ROWS, _zero, 0)

    def _group(t, _):
        w16 = lw_v[pl.ds(t * 16, 16)]
        d16 = ld_v[pl.ds(t * 16, 16)]
        for l in range(16):
            wv = jnp.full((16,), w16[l], jnp.float32)
            dl = d16[l]
            acc_v[dl, pl.ds(0, 16)] = acc_v[dl, pl.ds(0, 16)] + wv
        return 0

    lax.fori_loop(0, ng, _group, 0)

    def _wb(off, n):
        pltpu.sync_copy(acc_v.at[pl.ds(off, n)],
                        deg_hbm.at[pl.ds(wid * ROWS + off, n)])

    _wb(0, 128)
    _wb(128, 128)

    @pl.when(wid < NW - 1)
    def _():
        _wb(256, ROWS - 256)

    @pl.when(wid == NW - 1)
    def _():
        _wb(256, ROWS_LAST - 256)


_deg_call = pl.kernel(
    _deg_body,
    out_type=jax.ShapeDtypeStruct((N, 16), jnp.float32),
    mesh=_mesh,
    compiler_params=pltpu.CompilerParams(needs_layout_passes=False),
    scratch_types=[
        pltpu.VMEM((CAP + 16,), jnp.int32),
        pltpu.VMEM((CAP + 16,), jnp.float32),
        pltpu.VMEM((NW * 8 + 16,), jnp.int32),
        pltpu.VMEM((ACC_ROWS, 16), jnp.float32),
    ],
)


def _tc1_body(x_ref, sdeg_ref, w1_ref, dinv_ref, g_ref):
    deg = sdeg_ref[...][:, 0:1] + 1.0
    dinv = lax.rsqrt(deg)
    dinv_ref[...] = dinv
    h = jnp.dot(x_ref[...], w1_ref[...], preferred_element_type=jnp.float32)
    g_ref[...] = dinv * h


_tc1 = pl.pallas_call(
    _tc1_body,
    out_shape=(jax.ShapeDtypeStruct((N, 1), jnp.float32),
               jax.ShapeDtypeStruct((N, D), jnp.float32)),
)


def _bn_rows(h, gam, bet):
    mu = jnp.mean(h, axis=0, keepdims=True)
    var = jnp.mean(h * h, axis=0, keepdims=True) - mu * mu
    return (h - mu) * lax.rsqrt(var + 1e-5) * gam + bet


def _tc_mid_body(s_ref, g_ref, dinv_ref, b_ref, gam_ref, bet_ref, wn_ref,
                 gn_ref):
    h = dinv_ref[...] * (s_ref[...] + g_ref[...]) + b_ref[...]
    h = _bn_rows(h, gam_ref[...], bet_ref[...])
    h = jnp.maximum(h, 0.0)
    hw = jnp.dot(h, wn_ref[...], preferred_element_type=jnp.float32)
    gn_ref[...] = dinv_ref[...] * hw


_tc_mid = pl.pallas_call(
    _tc_mid_body,
    out_shape=jax.ShapeDtypeStruct((N, D), jnp.float32),
)


def _tc_final_body(s_ref, g_ref, dinv_ref, b_ref, gam_ref, bet_ref,
                   batch_ref, out_ref):
    h = dinv_ref[...] * (s_ref[...] + g_ref[...]) + b_ref[...]
    h = _bn_rows(h, gam_ref[...], bet_ref[...])
    gid = lax.broadcasted_iota(jnp.int32, (N, G), 1)
    oh = (batch_ref[...] == gid).astype(jnp.float32)
    sums = lax.dot_general(oh, h, (((0,), (0,)), ((), ())),
                           preferred_element_type=jnp.float32)
    cnt = jnp.sum(oh, axis=0, keepdims=True)
    inv = 1.0 / jnp.maximum(cnt, 1.0)
    sums = sums * lax.dot_general(inv, jnp.ones((1, D), jnp.float32),
                                  (((0,), (0,)), ((), ())),
                                  preferred_element_type=jnp.float32)
    out_ref[...] = sums


_tc_final = pl.pallas_call(
    _tc_final_body,
    out_shape=jax.ShapeDtypeStruct((G, D), jnp.float32),
)


def kernel(x, edge_index, edge_weight, batch, W1, b1, W2, b2, W3, b3,
           g1, be1, g2, be2, g3, be3):
    src = edge_index[0]
    dst = edge_index[1]
    b1r = b1.reshape(1, D); b2r = b2.reshape(1, D); b3r = b3.reshape(1, D)
    g1r = g1.reshape(1, D); g2r = g2.reshape(1, D); g3r = g3.reshape(1, D)
    be1r = be1.reshape(1, D); be2r = be2.reshape(1, D); be3r = be3.reshape(1, D)

    srcl, dstl, wl, cnts = _part_call(src, dst, edge_weight)
    sdeg = _deg_call(dstl, wl, cnts)
    dinv, g = _tc1(x, sdeg, W1)

    S = _agg_full(_to_bf16_table(g), srcl, dstl, wl, cnts)
    g = _tc_mid(S, g, dinv, b1r, g1r, be1r, W1)

    S = _agg_full(_to_bf16_table(g), srcl, dstl, wl, cnts)
    g = _tc_mid(S, g, dinv, b1r, g1r, be1r, W2)

    S = _agg_full(_to_bf16_table(g), srcl, dstl, wl, cnts)
    g = _tc_mid(S, g, dinv, b2r, g2r, be2r, W3)

    S = _agg_full(_to_bf16_table(g), srcl, dstl, wl, cnts)
    return _tc_final(S, g, dinv, b3r, g3r, be3r, batch.reshape(N, 1))


# speedup vs baseline: 1.2428x; 1.0047x over previous
import jax
import jax.numpy as jnp
from jax import lax
from jax.experimental import pallas as pl
from jax.experimental.pallas import tpu as pltpu
from jax.experimental.pallas import tpu_sc as plsc

N = 10000
E = 160000
D = 256
G = 64

NC = 2
NS = 16
NW = NC * NS
ROWS = 312
ROWS_LAST = N - (NW - 1) * ROWS
ACC_ROWS = 336
TRASH = 328
CAP = 7680
SCN = 16000
BLK = 128
PADB = 128

_PERM = []
for _p in range(D // 32):
    for _i in range(16):
        _PERM += [32 * _p + _i, 32 * _p + 16 + _i]

_mesh = plsc.VectorSubcoreMesh(core_axis_name="c", subcore_axis_name="s")


def _part_body(src_hbm, dst_hbm, w_hbm, srcl_hbm, dstl_hbm, wl_hbm, cnt_hbm,
               deg_hbm, s_sc, d_sc, w_sc, ls_v, ld_v, lw_v, cb_v, dacc_v):
    c = lax.axis_index("c")
    s = lax.axis_index("s")
    wid = c * NS + s
    lo = wid * ROWS
    hi = jnp.where(wid == NW - 1, N, lo + ROWS)
    lo_v = jnp.full((16,), lo, jnp.int32)
    hi_v = jnp.full((16,), hi, jnp.int32)

    def _chunk(ch, off):
        base = ch * SCN
        pltpu.sync_copy(src_hbm.at[pl.ds(base, SCN)], s_sc)
        pltpu.sync_copy(dst_hbm.at[pl.ds(base, SCN)], d_sc)
        pltpu.sync_copy(w_hbm.at[pl.ds(base, SCN)], w_sc)

        def _vec(i, off):
            sl = pl.ds(i * 16, 16)
            d16 = d_sc[sl]
            mask = (d16 >= lo_v) & (d16 < hi_v)
            inc = plsc.cumsum(mask.astype(jnp.int32))
            pos = off + inc - 1
            plsc.store_scatter(ld_v, [pos], d16 - lo_v, mask=mask)
            plsc.store_scatter(ls_v, [pos], s_sc[sl], mask=mask)
            plsc.store_scatter(lw_v, [pos], w_sc[sl], mask=mask)
            return off + inc[15]

        return lax.fori_loop(0, SCN // 16, _vec, off)

    cnt = lax.fori_loop(0, E // SCN, _chunk, jnp.int32(0))

    zi = jnp.zeros((16,), jnp.int32)
    zf = jnp.zeros((16,), jnp.float32)
    tv = jnp.full((16,), TRASH, jnp.int32)
    for t in range(PADB // 16):
        sl = pl.ds(cnt + t * 16, 16)
        ls_v[sl] = zi
        ld_v[sl] = tv
        lw_v[sl] = zf
    cntp = ((cnt + PADB - 1) // PADB) * PADB

    pltpu.sync_copy(ls_v.at[pl.ds(0, CAP)], srcl_hbm.at[pl.ds(wid * CAP, CAP)])
    pltpu.sync_copy(ld_v.at[pl.ds(0, CAP)], dstl_hbm.at[pl.ds(wid * CAP, CAP)])
    pltpu.sync_copy(lw_v.at[pl.ds(0, CAP)], wl_hbm.at[pl.ds(wid * CAP, CAP)])

    cb_v[pl.ds(0, 16)] = jnp.full((16,), cntp, jnp.int32)
    pltpu.sync_copy(cb_v.at[pl.ds(0, 8)], cnt_hbm.at[pl.ds(wid * 8, 8)])

    zf16 = jnp.zeros((16,), jnp.float32)

    def _dzero(i, _):
        dacc_v[i, pl.ds(0, 16)] = zf16
        return 0

    lax.fori_loop(0, ACC_ROWS, _dzero, 0)

    def _dgroup(t, _):
        w16 = lw_v[pl.ds(t * 16, 16)]
        d16 = ld_v[pl.ds(t * 16, 16)]
        for l in range(16):
            wv = jnp.full((16,), w16[l], jnp.float32)
            dl = d16[l]
            dacc_v[dl, pl.ds(0, 16)] = dacc_v[dl, pl.ds(0, 16)] + wv
        return 0

    lax.fori_loop(0, cntp // 16, _dgroup, 0)

    def _dwb(off, n):
        pltpu.sync_copy(dacc_v.at[pl.ds(off, n)],
                        deg_hbm.at[pl.ds(wid * ROWS + off, n)])

    _dwb(0, 128)
    _dwb(128, 128)

    @pl.when(wid < NW - 1)
    def _():
        _dwb(256, ROWS - 256)

    @pl.when(wid == NW - 1)
    def _():
        _dwb(256, ROWS_LAST - 256)


_part_call = pl.kernel(
    _part_body,
    out_type=(jax.ShapeDtypeStruct((NW * CAP,), jnp.int32),
              jax.ShapeDtypeStruct((NW * CAP,), jnp.int32),
              jax.ShapeDtypeStruct((NW * CAP,), jnp.float32),
              jax.ShapeDtypeStruct((NW * 8,), jnp.int32),
              jax.ShapeDtypeStruct((N, 16), jnp.float32)),
    mesh=_mesh,
    compiler_params=pltpu.CompilerParams(needs_layout_passes=False),
    scratch_types=[
        pltpu.VMEM((SCN,), jnp.int32),
        pltpu.VMEM((SCN,), jnp.int32),
        pltpu.VMEM((SCN,), jnp.float32),
        pltpu.VMEM((CAP + PADB,), jnp.int32),
        pltpu.VMEM((CAP + PADB,), jnp.int32),
        pltpu.VMEM((CAP + PADB,), jnp.float32),
        pltpu.VMEM((16,), jnp.int32),
        pltpu.VMEM((ACC_ROWS, 16), jnp.float32),
    ],
)


def _agg_body(gb_hbm, srcl_hbm, dstl_hbm, wl_hbm, cnt_hbm, s_hbm,
              ls_v, ld_v, lw_v, cv_v, rows_v, acc_v, sem):
    c = lax.axis_index("c")
    s = lax.axis_index("s")
    wid = c * NS + s

    pltpu.sync_copy(srcl_hbm.at[pl.ds(wid * CAP, CAP)],
                    ls_v.at[pl.ds(0, CAP)])
    pltpu.sync_copy(dstl_hbm.at[pl.ds(wid * CAP, CAP)],
                    ld_v.at[pl.ds(0, CAP)])
    pltpu.sync_copy(wl_hbm.at[pl.ds(wid * CAP, CAP)],
                    lw_v.at[pl.ds(0, CAP)])
    pltpu.sync_copy(cnt_hbm, cv_v.at[pl.ds(0, NW * 8)])
    nb = cv_v[pl.ds(wid * 8, 16)][0] // BLK

    zf = jnp.zeros((16,), jnp.float32)

    def _zero(i, _):
        for jj in range(D // 16):
            acc_v[i, pl.ds(jj * 16, 16)] = zf
        return 0

    lax.fori_loop(0, ACC_ROWS, _zero, 0)

    himask = jnp.full((16,), -65536, jnp.int32)

    def _block(b, _):
        pltpu.async_copy(gb_hbm.at[ls_v.at[pl.ds(b * BLK, BLK)]],
                         rows_v, sem).wait()

        def _group(t, _):
            e0 = b * BLK + t * 16
            w16 = lw_v[pl.ds(e0, 16)]
            d16 = ld_v[pl.ds(e0, 16)]
            for l in range(16):
                wv = jnp.full((16,), w16[l], jnp.float32)
                dl = d16[l]
                k = t * 16 + l
                for cc in range(D // 32):
                    u = rows_v[k, pl.ds(cc * 16, 16)]
                    flo = plsc.bitcast(u << 16, jnp.float32)
                    fhi = plsc.bitcast(u & himask, jnp.float32)
                    col = cc * 32
                    sl0 = pl.ds(col, 16)
                    sl1 = pl.ds(col + 16, 16)
                    acc_v[dl, sl0] = acc_v[dl, sl0] + flo * wv
                    acc_v[dl, sl1] = acc_v[dl, sl1] + fhi * wv
            return 0

        lax.fori_loop(0, BLK // 16, _group, 0)
        return 0

    lax.fori_loop(0, nb, _block, 0)

    def _wb(off, n):
        pltpu.sync_copy(acc_v.at[pl.ds(off, n)],
                        s_hbm.at[pl.ds(wid * ROWS + off, n)])

    _wb(0, 128)
    _wb(128, 128)

    @pl.when(wid < NW - 1)
    def _():
        _wb(256, ROWS - 256)

    @pl.when(wid == NW - 1)
    def _():
        _wb(256, ROWS_LAST - 256)


_agg_full = pl.kernel(
    _agg_body,
    out_type=jax.ShapeDtypeStruct((N, D), jnp.float32),
    mesh=_mesh,
    compiler_params=pltpu.CompilerParams(needs_layout_passes=False),
    scratch_types=[
        pltpu.VMEM((CAP + 16,), jnp.int32),
        pltpu.VMEM((CAP + 16,), jnp.int32),
        pltpu.VMEM((CAP + 16,), jnp.float32),
        pltpu.VMEM((NW * 8 + 16,), jnp.int32),
        pltpu.VMEM((BLK, D // 2), jnp.int32),
        pltpu.VMEM((ACC_ROWS, D), jnp.float32),
        pltpu.SemaphoreType.DMA,
    ],
)


def _to_bf16_table(g):
    gb = g.astype(jnp.bfloat16)[:, jnp.array(_PERM, dtype=jnp.int32)]
    return lax.bitcast_convert_type(gb.reshape(N, D // 2, 2), jnp.int32)


def _tc1_body(x_ref, sdeg_ref, w1_ref, dinv_ref, g_ref):
    deg = sdeg_ref[...][:, 0:1] + 1.0
    dinv = lax.rsqrt(deg)
    dinv_ref[...] = dinv
    h = jnp.dot(x_ref[...], w1_ref[...], preferred_element_type=jnp.float32)
    g_ref[...] = dinv * h


_tc1 = pl.pallas_call(
    _tc1_body,
    out_shape=(jax.ShapeDtypeStruct((N, 1), jnp.float32),
               jax.ShapeDtypeStruct((N, D), jnp.float32)),
)


def _bn_rows(h, gam, bet):
    mu = jnp.mean(h, axis=0, keepdims=True)
    var = jnp.mean(h * h, axis=0, keepdims=True) - mu * mu
    return (h - mu) * lax.rsqrt(var + 1e-5) * gam + bet


def _tc_mid_body(s_ref, g_ref, dinv_ref, b_ref, gam_ref, bet_ref, wn_ref,
                 gn_ref):
    h = dinv_ref[...] * (s_ref[...] + g_ref[...]) + b_ref[...]
    h = _bn_rows(h, gam_ref[...], bet_ref[...])
    h = jnp.maximum(h, 0.0)
    hw = jnp.dot(h, wn_ref[...], preferred_element_type=jnp.float32)
    gn_ref[...] = dinv_ref[...] * hw


_tc_mid = pl.pallas_call(
    _tc_mid_body,
    out_shape=jax.ShapeDtypeStruct((N, D), jnp.float32),
)


def _tc_final_body(s_ref, g_ref, dinv_ref, b_ref, gam_ref, bet_ref,
                   batch_ref, out_ref):
    h = dinv_ref[...] * (s_ref[...] + g_ref[...]) + b_ref[...]
    h = _bn_rows(h, gam_ref[...], bet_ref[...])
    gid = lax.broadcasted_iota(jnp.int32, (N, G), 1)
    oh = (batch_ref[...] == gid).astype(jnp.float32)
    sums = lax.dot_general(oh, h, (((0,), (0,)), ((), ())),
                           preferred_element_type=jnp.float32)
    cnt = jnp.sum(oh, axis=0, keepdims=True)
    inv = 1.0 / jnp.maximum(cnt, 1.0)
    sums = sums * lax.dot_general(inv, jnp.ones((1, D), jnp.float32),
                                  (((0,), (0,)), ((), ())),
                                  preferred_element_type=jnp.float32)
    out_ref[...] = sums


_tc_final = pl.pallas_call(
    _tc_final_body,
    out_shape=jax.ShapeDtypeStruct((G, D), jnp.float32),
)


def kernel(x, edge_index, edge_weight, batch, W1, b1, W2, b2, W3, b3,
           g1, be1, g2, be2, g3, be3):
    src = edge_index[0]
    dst = edge_index[1]
    b1r = b1.reshape(1, D); b2r = b2.reshape(1, D); b3r = b3.reshape(1, D)
    g1r = g1.reshape(1, D); g2r = g2.reshape(1, D); g3r = g3.reshape(1, D)
    be1r = be1.reshape(1, D); be2r = be2.reshape(1, D); be3r = be3.reshape(1, D)

    srcl, dstl, wl, cnts, sdeg = _part_call(src, dst, edge_weight)
    dinv, g = _tc1(x, sdeg, W1)

    S = _agg_full(_to_bf16_table(g), srcl, dstl, wl, cnts)
    g = _tc_mid(S, g, dinv, b1r, g1r, be1r, W1)

    S = _agg_full(_to_bf16_table(g), srcl, dstl, wl, cnts)
    g = _tc_mid(S, g, dinv, b1r, g1r, be1r, W2)

    S = _agg_full(_to_bf16_table(g), srcl, dstl, wl, cnts)
    g = _tc_mid(S, g, dinv, b2r, g2r, be2r, W3)

    S = _agg_full(_to_bf16_table(g), srcl, dstl, wl, cnts)
    return _tc_final(S, g, dinv, b3r, g3r, be3r, batch.reshape(N, 1))
